# baseline (speedup 1.0000x reference)
import functools

import jax
import jax.numpy as jnp
from jax import lax
from jax.experimental import pallas as pl
from jax.experimental.pallas import tpu as pltpu
from jax.experimental.pallas import tpu_sc as plsc

N_CLS = 1000
N_CTX = 16
D = 512
SEQ = 77
SUF = 60
B = 1024
ROW = SEQ * D

NC, NS = 2, 16
NW = NC * NS
BPW = B // NW
C = 8
NCHUNK = BPW // C
SUF_Q = 6
SUF_R = SUF // SUF_Q
TP_PAD = 128

PREF_OFF = 0
CTX_OFF = D
SUF_OFF = (1 + N_CTX) * D


def _body(cls_hbm, ctx_hbm, pref_hbm, suf_hbm, tp_hbm,
          out_p, out_tp,
          idx_v, suf_idx, pref_v, ctx_v, suf_v, tp_v,
          gsem, wsem, qsem):
    wid = lax.axis_index("s") * NC + lax.axis_index("c")
    base = wid * BPW

    pltpu.sync_copy(cls_hbm.at[pl.ds(base, BPW)], idx_v)
    for h in range(BPW // 16):
        v = idx_v[pl.ds(h * 16, 16)] * SUF_Q
        for q in range(SUF_Q):
            suf_idx[q, pl.ds(h * 16, 16)] = v + q

    for j in range(NCHUNK):
        i8 = idx_v.at[pl.ds(j * C, C)]
        row0 = base + j * C
        cp_p = pltpu.async_copy(pref_hbm.at[i8], pref_v, gsem)
        cp_c = pltpu.async_copy(ctx_hbm.at[i8], ctx_v, gsem)
        cp_t = pltpu.async_copy(tp_hbm.at[i8], tp_v, gsem)
        cp_p.wait()
        cp_c.wait()
        cp_t.wait()
        w_p = pltpu.async_copy(
            pref_v, out_p.at[pl.ds(row0, C), pl.ds(PREF_OFF, D)], wsem)
        w_c = pltpu.async_copy(
            ctx_v, out_p.at[pl.ds(row0, C), pl.ds(CTX_OFF, N_CTX * D)], wsem)
        w_t = pltpu.async_copy(tp_v, out_tp.at[pl.ds(row0, C)], wsem)
        for q in range(SUF_Q):
            sq = suf_idx.at[q, pl.ds(j * C, C)]
            g = pltpu.async_copy(suf_hbm.at[sq], suf_v, gsem)
            g.wait()
            w_s = pltpu.async_copy(
                suf_v,
                out_p.at[pl.ds(row0, C), pl.ds(SUF_OFF + q * SUF_R * D, SUF_R * D)],
                qsem)
            w_s.wait()
        w_p.wait()
        w_c.wait()
        w_t.wait()


@functools.partial(
    pl.kernel,
    out_type=(jax.ShapeDtypeStruct((B, ROW), jnp.float32),
              jax.ShapeDtypeStruct((B, TP_PAD), jnp.int32)),
    mesh=plsc.VectorSubcoreMesh(core_axis_name="c", subcore_axis_name="s",
                                num_cores=NC, num_subcores=NS),
    scratch_types=[
        pltpu.VMEM((BPW,), jnp.int32),
        pltpu.VMEM((SUF_Q, BPW), jnp.int32),
        pltpu.VMEM((C, D), jnp.float32),
        pltpu.VMEM((C, N_CTX * D), jnp.float32),
        pltpu.VMEM((C, SUF_R * D), jnp.float32),
        pltpu.VMEM((C, TP_PAD), jnp.int32),
        pltpu.SemaphoreType.DMA,
        pltpu.SemaphoreType.DMA,
        pltpu.SemaphoreType.DMA,
    ],
)
def _sc_gather(cls_hbm, ctx_hbm, pref_hbm, suf_hbm, tp_hbm, out_p, out_tp,
               idx_v, suf_idx, pref_v, ctx_v, suf_v, tp_v, gsem, wsem, qsem):
    _body(cls_hbm, ctx_hbm, pref_hbm, suf_hbm, tp_hbm, out_p, out_tp,
          idx_v, suf_idx, pref_v, ctx_v, suf_v, tp_v, gsem, wsem, qsem)


def kernel(cls_id, ctx, token_prefix, token_suffix, tokenized_prompts):
    cls32 = cls_id.astype(jnp.int32)
    ctx2d = ctx.reshape(N_CLS, N_CTX * D)
    pref2d = token_prefix.reshape(N_CLS, D)
    suf2d = token_suffix.reshape(N_CLS * SUF_Q, SUF_R * D)
    tp_pad = jnp.pad(tokenized_prompts, ((0, 0), (0, TP_PAD - SEQ)))
    prompts, tp = _sc_gather(cls32, ctx2d, pref2d, suf2d, tp_pad)
    return prompts.reshape(B, SEQ, D), tp[:, :SEQ]


# speedup vs baseline: 1.4218x; 1.4218x over previous
import functools

import jax
import jax.numpy as jnp
from jax import lax
from jax.experimental import pallas as pl
from jax.experimental.pallas import tpu as pltpu
from jax.experimental.pallas import tpu_sc as plsc

N_CLS = 1000
N_CTX = 16
D = 512
SEQ = 77
SUF = 60
SUF_BULK = 56
TAIL = SUF - SUF_BULK
B = 1024
LANES = D // 16

NC, NS = 2, 16
NW = NC * NS
BPW = B // NW
TP_PAD = 128


def _body(cls_hbm, clsr_hbm, ctx_hbm, pref_hbm, suf_hbm, tail_hbm, tp_hbm,
          out_p, out_tp,
          idx_v, idxx8, asm, pref_st, tail_st, tp_v,
          gsem, wsem, tsem):
    wid = lax.axis_index("s") * NC + lax.axis_index("c")
    base = wid * BPW

    pltpu.sync_copy(cls_hbm.at[pl.ds(base, BPW)], idx_v)
    pltpu.sync_copy(clsr_hbm.at[pl.ds(8 * base, 8 * BPW)], idxx8)

    tcp = pltpu.async_copy(tp_hbm.at[idx_v], tp_v, tsem)

    def element(i):
        off = pl.multiple_of(i * 8, 8)
        i1 = idxx8.at[pl.ds(off, 1)]
        g1 = pltpu.async_copy(ctx_hbm.at[i1], asm.at[:, pl.ds(0, N_CTX), :], gsem)
        g2 = pltpu.async_copy(suf_hbm.at[i1, pl.ds(0, SUF_BULK), :],
                              asm.at[:, pl.ds(N_CTX, SUF_BULK), :], gsem)
        g3 = pltpu.async_copy(pref_hbm.at[i1], pref_st, gsem)
        g4 = pltpu.async_copy(tail_hbm.at[i1], tail_st, gsem)
        g1.wait()
        g2.wait()
        g3.wait()
        g4.wait()

        def shift(t, carry):
            d = (N_CTX + SUF_BULK) - t
            for c in range(LANES):
                asm[0, d, pl.ds(c * 16, 16)] = asm[0, d - 1, pl.ds(c * 16, 16)]
            return carry

        lax.fori_loop(0, N_CTX + SUF_BULK, shift, 0, unroll=2)

        for c in range(LANES):
            asm[0, 0, pl.ds(c * 16, 16)] = pref_st[0, 0, pl.ds(c * 16, 16)]
        for r in range(TAIL):
            for c in range(LANES):
                asm[0, 1 + N_CTX + SUF_BULK + r, pl.ds(c * 16, 16)] = (
                    tail_st[0, r, pl.ds(c * 16, 16)])

        pltpu.async_copy(asm, out_p.at[pl.ds(base + i, 1)], wsem).wait()

    lax.fori_loop(0, BPW, lambda i, c: (element(i), c)[1], 0)

    tcp.wait()
    pltpu.sync_copy(tp_v, out_tp.at[pl.ds(base, BPW)])


@functools.partial(
    pl.kernel,
    out_type=(jax.ShapeDtypeStruct((B, SEQ, D), jnp.float32),
              jax.ShapeDtypeStruct((B, TP_PAD), jnp.int32)),
    mesh=plsc.VectorSubcoreMesh(core_axis_name="c", subcore_axis_name="s",
                                num_cores=NC, num_subcores=NS),
    scratch_types=[
        pltpu.VMEM((BPW,), jnp.int32),
        pltpu.VMEM((8 * BPW,), jnp.int32),
        pltpu.VMEM((1, SEQ, D), jnp.float32),
        pltpu.VMEM((1, 1, D), jnp.float32),
        pltpu.VMEM((1, TAIL, D), jnp.float32),
        pltpu.VMEM((BPW, TP_PAD), jnp.int32),
        pltpu.SemaphoreType.DMA,
        pltpu.SemaphoreType.DMA,
        pltpu.SemaphoreType.DMA,
    ],
)
def _sc_gather(cls_hbm, clsr_hbm, ctx_hbm, pref_hbm, suf_hbm, tail_hbm, tp_hbm,
               out_p, out_tp,
               idx_v, idxx8, asm, pref_st, tail_st, tp_v, gsem, wsem, tsem):
    _body(cls_hbm, clsr_hbm, ctx_hbm, pref_hbm, suf_hbm, tail_hbm, tp_hbm,
          out_p, out_tp,
          idx_v, idxx8, asm, pref_st, tail_st, tp_v, gsem, wsem, tsem)


def kernel(cls_id, ctx, token_prefix, token_suffix, tokenized_prompts):
    cls32 = cls_id.astype(jnp.int32)
    cls_rep8 = jnp.repeat(cls32, 8)
    tail_tab = token_suffix[:, SUF_BULK:, :]
    tp_pad = jnp.pad(tokenized_prompts, ((0, 0), (0, TP_PAD - SEQ)))
    prompts, tp = _sc_gather(cls32, cls_rep8, ctx, token_prefix, token_suffix,
                             tail_tab, tp_pad)
    return prompts, tp[:, :SEQ]


# speedup vs baseline: 1.6150x; 1.1359x over previous
import functools

import jax
import jax.numpy as jnp
from jax import lax
from jax.experimental import pallas as pl
from jax.experimental.pallas import tpu as pltpu
from jax.experimental.pallas import tpu_sc as plsc

N_CLS = 1000
N_CTX = 16
D = 512
SEQ = 77
SUF = 60
SUF_BULK = 56
TAIL = SUF - SUF_BULK
B = 1024
LANES = D // 16
SHIFT_ROWS = N_CTX + SUF_BULK

NC, NS = 2, 16
NW = NC * NS
BPW = B // NW
TP_PAD = 128


def _body(cls_hbm, clsr_hbm, ctx_hbm, pref_hbm, suf_hbm, tail_hbm, tp_hbm,
          out_p, out_tp,
          idx_v, idxx8, asm0, asm1, pref0, pref1, tail0, tail1, tp_v,
          g0, g1, w0, w1, tsem):
    wid = lax.axis_index("s") * NC + lax.axis_index("c")
    base = wid * BPW
    asms, prefs, tails = (asm0, asm1), (pref0, pref1), (tail0, tail1)
    gsems, wsems = (g0, g1), (w0, w1)

    pltpu.sync_copy(cls_hbm.at[pl.ds(base, BPW)], idx_v)
    pltpu.sync_copy(clsr_hbm.at[pl.ds(8 * base, 8 * BPW)], idxx8)

    tcp = pltpu.async_copy(tp_hbm.at[idx_v], tp_v, tsem)

    def g_copies(i, s, issue):
        off = pl.multiple_of(i * 8, 8)
        i1 = idxx8.at[pl.ds(off, 1)]
        mk = pltpu.async_copy if issue else (
            lambda a, b, c: pltpu.make_async_copy(a, b, c))
        return (
            mk(ctx_hbm.at[i1], asms[s].at[:, pl.ds(0, N_CTX), :], gsems[s]),
            mk(suf_hbm.at[i1, pl.ds(0, SUF_BULK), :],
               asms[s].at[:, pl.ds(N_CTX, SUF_BULK), :], gsems[s]),
            mk(pref_hbm.at[i1], prefs[s], gsems[s]),
            mk(tail_hbm.at[i1], tails[s], gsems[s]),
        )

    def wait_g(i, s):
        for dsc in g_copies(i, s, issue=False):
            dsc.wait()

    def issue_w(i, s):
        return pltpu.async_copy(asms[s], out_p.at[pl.ds(base + i, 1)], wsems[s])

    def wait_w(i, s):
        pltpu.make_async_copy(asms[s], out_p.at[pl.ds(base + i, 1)],
                              wsems[s]).wait()

    def assemble(i, s):
        asm, pref_st, tail_st = asms[s], prefs[s], tails[s]

        def shift(t, carry):
            d = SHIFT_ROWS - t
            for c in range(LANES):
                asm[0, d, pl.ds(c * 16, 16)] = asm[0, d - 1, pl.ds(c * 16, 16)]
            return carry

        lax.fori_loop(0, SHIFT_ROWS, shift, 0, unroll=2)

        for c in range(LANES):
            asm[0, 0, pl.ds(c * 16, 16)] = pref_st[0, 0, pl.ds(c * 16, 16)]
        for r in range(TAIL):
            for c in range(LANES):
                asm[0, 1 + SHIFT_ROWS + r, pl.ds(c * 16, 16)] = (
                    tail_st[0, r, pl.ds(c * 16, 16)])

    g_copies(0, 0, issue=True)

    def group(g, carry):
        i0 = 2 * g
        i1 = i0 + 1
        wait_g(i0, 0)

        @pl.when(g > 0)
        def _():
            wait_w(i1 - 2, 1)

        g_copies(i1, 1, issue=True)
        assemble(i0, 0)
        issue_w(i0, 0)
        wait_g(i1, 1)
        assemble(i1, 1)
        issue_w(i1, 1)
        wait_w(i0, 0)

        @pl.when(g < BPW // 2 - 1)
        def _():
            g_copies(i0 + 2, 0, issue=True)

        return carry

    lax.fori_loop(0, BPW // 2, group, 0)
    wait_w(BPW - 1, 1)

    tcp.wait()
    pltpu.sync_copy(tp_v, out_tp.at[pl.ds(base, BPW)])


@functools.partial(
    pl.kernel,
    out_type=(jax.ShapeDtypeStruct((B, SEQ, D), jnp.float32),
              jax.ShapeDtypeStruct((B, TP_PAD), jnp.int32)),
    mesh=plsc.VectorSubcoreMesh(core_axis_name="c", subcore_axis_name="s",
                                num_cores=NC, num_subcores=NS),
    compiler_params=pltpu.CompilerParams(disable_bounds_checks=True),
    scratch_types=[
        pltpu.VMEM((BPW,), jnp.int32),
        pltpu.VMEM((8 * BPW,), jnp.int32),
        pltpu.VMEM((1, SEQ, D), jnp.float32),
        pltpu.VMEM((1, SEQ, D), jnp.float32),
        pltpu.VMEM((1, 1, D), jnp.float32),
        pltpu.VMEM((1, 1, D), jnp.float32),
        pltpu.VMEM((1, TAIL, D), jnp.float32),
        pltpu.VMEM((1, TAIL, D), jnp.float32),
        pltpu.VMEM((BPW, TP_PAD), jnp.int32),
        pltpu.SemaphoreType.DMA,
        pltpu.SemaphoreType.DMA,
        pltpu.SemaphoreType.DMA,
        pltpu.SemaphoreType.DMA,
        pltpu.SemaphoreType.DMA,
    ],
)
def _sc_gather(cls_hbm, clsr_hbm, ctx_hbm, pref_hbm, suf_hbm, tail_hbm, tp_hbm,
               out_p, out_tp,
               idx_v, idxx8, asm0, asm1, pref0, pref1, tail0, tail1, tp_v,
               g0, g1, w0, w1, tsem):
    _body(cls_hbm, clsr_hbm, ctx_hbm, pref_hbm, suf_hbm, tail_hbm, tp_hbm,
          out_p, out_tp,
          idx_v, idxx8, asm0, asm1, pref0, pref1, tail0, tail1, tp_v,
          g0, g1, w0, w1, tsem)


def kernel(cls_id, ctx, token_prefix, token_suffix, tokenized_prompts):
    cls32 = cls_id.astype(jnp.int32)
    cls_rep8 = jnp.repeat(cls32, 8)
    tail_tab = token_suffix[:, SUF_BULK:, :]
    tp_pad = jnp.pad(tokenized_prompts, ((0, 0), (0, TP_PAD - SEQ)))
    prompts, tp = _sc_gather(cls32, cls_rep8, ctx, token_prefix, token_suffix,
                             tail_tab, tp_pad)
    return prompts, tp[:, :SEQ]


# speedup vs baseline: 6.2387x; 3.8630x over previous
import functools

import jax
import jax.numpy as jnp
from jax import lax
from jax.experimental import pallas as pl
from jax.experimental.pallas import tpu as pltpu
from jax.experimental.pallas import tpu_sc as plsc

N_CLS = 1000
N_CTX = 16
D = 512
SEQ = 77
SUF = 60
B = 1024

NC, NS = 2, 16
NW = NC * NS
BPW = B // NW
TP_PAD = 128


def _body(cls_hbm, ctx2_hbm, pref_hbm, suf2_hbm, tp_hbm,
          out_pT, out_tp,
          idx_v, sb0, sb1, pref_st, tp_v,
          g0, g1, w0, w1, psem, tsem):
    wid = lax.axis_index("s") * NC + lax.axis_index("c")
    base = wid * BPW
    sbs = (sb0, sb1)
    gsems = (g0, g1)
    wsems = (w0, w1)

    pltpu.sync_copy(cls_hbm.at[pl.ds(base, BPW)], idx_v)

    tcp = pltpu.async_copy(tp_hbm.at[idx_v], tp_v, tsem)
    pcp = pltpu.async_copy(pref_hbm.at[idx_v], pref_st, psem)

    ida = idx_v[pl.ds(0, 16)]
    idb = idx_v[pl.ds(16, 16)]

    def slab_gather(t, s):
        if t < N_CTX:
            tab, ra, rb = ctx2_hbm, ida * N_CTX + t, idb * N_CTX + t
        else:
            r = t - N_CTX
            tab, ra, rb = suf2_hbm, ida + r * N_CLS, idb + r * N_CLS
        return (
            pltpu.async_copy(tab.at[ra], sbs[s].at[pl.ds(0, 16), :], gsems[s]),
            pltpu.async_copy(tab.at[rb], sbs[s].at[pl.ds(16, 16), :], gsems[s]),
        )

    def slab_write(t, s):
        return pltpu.async_copy(sbs[s], out_pT.at[t + 1, pl.ds(base, BPW), :],
                                wsems[s])

    NSLAB = SEQ - 1
    g = [None] * NSLAB
    w = [None] * NSLAB
    g[0] = slab_gather(0, 0)
    for t in range(NSLAB):
        s = t % 2
        if t + 1 < NSLAB:
            if t >= 1:
                w[t - 1].wait()
            g[t + 1] = slab_gather(t + 1, (t + 1) % 2)
        for dsc in g[t]:
            dsc.wait()
        w[t] = slab_write(t, s)
    w[NSLAB - 2].wait()
    w[NSLAB - 1].wait()

    pcp.wait()
    pltpu.sync_copy(pref_st.at[:, 0, :], out_pT.at[0, pl.ds(base, BPW), :])
    tcp.wait()
    pltpu.sync_copy(tp_v, out_tp.at[pl.ds(base, BPW)])


@functools.partial(
    pl.kernel,
    out_type=(jax.ShapeDtypeStruct((SEQ, B, D), jnp.float32),
              jax.ShapeDtypeStruct((B, TP_PAD), jnp.int32)),
    mesh=plsc.VectorSubcoreMesh(core_axis_name="c", subcore_axis_name="s",
                                num_cores=NC, num_subcores=NS),
    compiler_params=pltpu.CompilerParams(disable_bounds_checks=True),
    scratch_types=[
        pltpu.VMEM((BPW,), jnp.int32),
        pltpu.VMEM((BPW, D), jnp.float32),
        pltpu.VMEM((BPW, D), jnp.float32),
        pltpu.VMEM((BPW, 1, D), jnp.float32),
        pltpu.VMEM((BPW, TP_PAD), jnp.int32),
        pltpu.SemaphoreType.DMA,
        pltpu.SemaphoreType.DMA,
        pltpu.SemaphoreType.DMA,
        pltpu.SemaphoreType.DMA,
        pltpu.SemaphoreType.DMA,
        pltpu.SemaphoreType.DMA,
    ],
)
def _sc_gather(cls_hbm, ctx2_hbm, pref_hbm, suf2_hbm, tp_hbm,
               out_pT, out_tp,
               idx_v, sb0, sb1, pref_st, tp_v,
               g0, g1, w0, w1, psem, tsem):
    _body(cls_hbm, ctx2_hbm, pref_hbm, suf2_hbm, tp_hbm, out_pT, out_tp,
          idx_v, sb0, sb1, pref_st, tp_v, g0, g1, w0, w1, psem, tsem)


def kernel(cls_id, ctx, token_prefix, token_suffix, tokenized_prompts):
    cls32 = cls_id.astype(jnp.int32)
    ctx2 = ctx.reshape(N_CLS * N_CTX, D)
    suf2 = jnp.transpose(token_suffix, (1, 0, 2)).reshape(SUF * N_CLS, D)
    tp_pad = jnp.pad(tokenized_prompts, ((0, 0), (0, TP_PAD - SEQ)))
    prompts_T, tp = _sc_gather(cls32, ctx2, token_prefix, suf2, tp_pad)
    return jnp.transpose(prompts_T, (1, 0, 2)), tp[:, :SEQ]


# speedup vs baseline: 6.5065x; 1.0429x over previous
import functools

import jax
import jax.numpy as jnp
from jax import lax
from jax.experimental import pallas as pl
from jax.experimental.pallas import tpu as pltpu
from jax.experimental.pallas import tpu_sc as plsc

N_CLS = 1000
N_CTX = 16
D = 512
SEQ = 77
SUF = 60
B = 1024

NC, NS = 2, 16
NW = NC * NS
BPW = B // NW
TP_PAD = 128
NSLOT = 4


def _body(cls_hbm, ctx2_hbm, pref_hbm, suf2_hbm, tp_hbm,
          out_pT, out_tp,
          idx_v, sb0, sb1, sb2, sb3, pref_st, tp_v,
          g0, g1, g2, g3, w0, w1, w2, w3, psem, tsem):
    wid = lax.axis_index("s") * NC + lax.axis_index("c")
    base = wid * BPW
    sbs = (sb0, sb1, sb2, sb3)
    gsems = (g0, g1, g2, g3)
    wsems = (w0, w1, w2, w3)

    pltpu.sync_copy(cls_hbm.at[pl.ds(base, BPW)], idx_v)

    tcp = pltpu.async_copy(tp_hbm.at[idx_v], tp_v, tsem)
    pcp = pltpu.async_copy(pref_hbm.at[idx_v], pref_st, psem)

    ida = idx_v[pl.ds(0, 16)]
    idb = idx_v[pl.ds(16, 16)]

    def slab_gather(t, s):
        if t < N_CTX:
            tab, ra, rb = ctx2_hbm, ida * N_CTX + t, idb * N_CTX + t
        else:
            r = t - N_CTX
            tab, ra, rb = suf2_hbm, ida + r * N_CLS, idb + r * N_CLS
        return (
            pltpu.async_copy(tab.at[ra], sbs[s].at[pl.ds(0, 16), :], gsems[s]),
            pltpu.async_copy(tab.at[rb], sbs[s].at[pl.ds(16, 16), :], gsems[s]),
        )

    def slab_write(t, s):
        return pltpu.async_copy(sbs[s], out_pT.at[t + 1, pl.ds(base, BPW), :],
                                wsems[s])

    NSLAB = SEQ - 1
    g = [None] * NSLAB
    w = [None] * NSLAB
    for t in range(NSLOT - 1):
        g[t] = slab_gather(t, t % NSLOT)
    for t in range(NSLAB):
        s = t % NSLOT
        nxt = t + NSLOT - 1
        if nxt < NSLAB:
            if nxt - NSLOT >= 0:
                w[nxt - NSLOT].wait()
            g[nxt] = slab_gather(nxt, nxt % NSLOT)
        for dsc in g[t]:
            dsc.wait()
        w[t] = slab_write(t, s)
    for t in range(max(0, NSLAB - NSLOT), NSLAB):
        if w[t] is not None and t >= NSLAB - NSLOT:
            w[t].wait()

    pcp.wait()
    pltpu.sync_copy(pref_st.at[:, 0, :], out_pT.at[0, pl.ds(base, BPW), :])
    tcp.wait()
    pltpu.sync_copy(tp_v, out_tp.at[pl.ds(base, BPW)])


@functools.partial(
    pl.kernel,
    out_type=(jax.ShapeDtypeStruct((SEQ, B, D), jnp.float32),
              jax.ShapeDtypeStruct((B, TP_PAD), jnp.int32)),
    mesh=plsc.VectorSubcoreMesh(core_axis_name="c", subcore_axis_name="s",
                                num_cores=NC, num_subcores=NS),
    compiler_params=pltpu.CompilerParams(disable_bounds_checks=True),
    scratch_types=[
        pltpu.VMEM((BPW,), jnp.int32),
        pltpu.VMEM((BPW, D), jnp.float32),
        pltpu.VMEM((BPW, D), jnp.float32),
        pltpu.VMEM((BPW, D), jnp.float32),
        pltpu.VMEM((BPW, D), jnp.float32),
        pltpu.VMEM((BPW, 1, D), jnp.float32),
        pltpu.VMEM((BPW, TP_PAD), jnp.int32),
        pltpu.SemaphoreType.DMA,
        pltpu.SemaphoreType.DMA,
        pltpu.SemaphoreType.DMA,
        pltpu.SemaphoreType.DMA,
        pltpu.SemaphoreType.DMA,
        pltpu.SemaphoreType.DMA,
        pltpu.SemaphoreType.DMA,
        pltpu.SemaphoreType.DMA,
        pltpu.SemaphoreType.DMA,
        pltpu.SemaphoreType.DMA,
    ],
)
def _sc_gather(cls_hbm, ctx2_hbm, pref_hbm, suf2_hbm, tp_hbm,
               out_pT, out_tp,
               idx_v, sb0, sb1, sb2, sb3, pref_st, tp_v,
               g0, g1, g2, g3, w0, w1, w2, w3, psem, tsem):
    _body(cls_hbm, ctx2_hbm, pref_hbm, suf2_hbm, tp_hbm, out_pT, out_tp,
          idx_v, sb0, sb1, sb2, sb3, pref_st, tp_v,
          g0, g1, g2, g3, w0, w1, w2, w3, psem, tsem)


def kernel(cls_id, ctx, token_prefix, token_suffix, tokenized_prompts):
    cls32 = cls_id.astype(jnp.int32)
    ctx2 = ctx.reshape(N_CLS * N_CTX, D)
    suf2 = jnp.transpose(token_suffix, (1, 0, 2)).reshape(SUF * N_CLS, D)
    tp_pad = jnp.pad(tokenized_prompts, ((0, 0), (0, TP_PAD - SEQ)))
    prompts_T, tp = _sc_gather(cls32, ctx2, token_prefix, suf2, tp_pad)
    return jnp.transpose(prompts_T, (1, 0, 2)), tp[:, :SEQ]


# speedup vs baseline: 6.5449x; 1.0059x over previous
import functools

import jax
import jax.numpy as jnp
from jax import lax
from jax.experimental import pallas as pl
from jax.experimental.pallas import tpu as pltpu
from jax.experimental.pallas import tpu_sc as plsc

N_CLS = 1000
N_CTX = 16
D = 512
SEQ = 77
SUF = 60
B = 1024

NC, NS = 2, 16
NW = NC * NS
BPW = B // NW
TP_PAD = 128
NSLOT = 6


def _body(cls_hbm, ctx2_hbm, pref_hbm, suf2_hbm, tp_hbm,
          out_pT, out_tp,
          idx_v, sb0, sb1, sb2, sb3, sb4, sb5, pref_st, tp_v,
          g0, g1, g2, g3, g4, g5, w0, w1, w2, w3, w4, w5, psem, tsem):
    wid = lax.axis_index("s") * NC + lax.axis_index("c")
    base = wid * BPW
    sbs = (sb0, sb1, sb2, sb3, sb4, sb5)
    gsems = (g0, g1, g2, g3, g4, g5)
    wsems = (w0, w1, w2, w3, w4, w5)

    pltpu.sync_copy(cls_hbm.at[pl.ds(base, BPW)], idx_v)

    tcp = pltpu.async_copy(tp_hbm.at[idx_v], tp_v, tsem)
    pcp = pltpu.async_copy(pref_hbm.at[idx_v], pref_st, psem)

    ida = idx_v[pl.ds(0, 16)]
    idb = idx_v[pl.ds(16, 16)]

    def slab_gather(t, s):
        if t < N_CTX:
            tab, ra, rb = ctx2_hbm, ida * N_CTX + t, idb * N_CTX + t
        else:
            r = t - N_CTX
            tab, ra, rb = suf2_hbm, ida + r * N_CLS, idb + r * N_CLS
        return (
            pltpu.async_copy(tab.at[ra], sbs[s].at[pl.ds(0, 16), :], gsems[s]),
            pltpu.async_copy(tab.at[rb], sbs[s].at[pl.ds(16, 16), :], gsems[s]),
        )

    def slab_write(t, s):
        return pltpu.async_copy(sbs[s], out_pT.at[t + 1, pl.ds(base, BPW), :],
                                wsems[s])

    NSLAB = SEQ - 1
    g = [None] * NSLAB
    w = [None] * NSLAB
    for t in range(NSLOT - 1):
        g[t] = slab_gather(t, t % NSLOT)
    for t in range(NSLAB):
        s = t % NSLOT
        nxt = t + NSLOT - 1
        if nxt < NSLAB:
            if nxt - NSLOT >= 0:
                w[nxt - NSLOT].wait()
            g[nxt] = slab_gather(nxt, nxt % NSLOT)
        for dsc in g[t]:
            dsc.wait()
        w[t] = slab_write(t, s)
    for t in range(max(0, NSLAB - NSLOT), NSLAB):
        if w[t] is not None and t >= NSLAB - NSLOT:
            w[t].wait()

    pcp.wait()
    pltpu.sync_copy(pref_st.at[:, 0, :], out_pT.at[0, pl.ds(base, BPW), :])
    tcp.wait()
    pltpu.sync_copy(tp_v, out_tp.at[pl.ds(base, BPW)])


@functools.partial(
    pl.kernel,
    out_type=(jax.ShapeDtypeStruct((SEQ, B, D), jnp.float32),
              jax.ShapeDtypeStruct((B, TP_PAD), jnp.int32)),
    mesh=plsc.VectorSubcoreMesh(core_axis_name="c", subcore_axis_name="s",
                                num_cores=NC, num_subcores=NS),
    compiler_params=pltpu.CompilerParams(disable_bounds_checks=True),
    scratch_types=[
        pltpu.VMEM((BPW,), jnp.int32),
        pltpu.VMEM((BPW, D), jnp.float32),
        pltpu.VMEM((BPW, D), jnp.float32),
        pltpu.VMEM((BPW, D), jnp.float32),
        pltpu.VMEM((BPW, D), jnp.float32),
        pltpu.VMEM((BPW, D), jnp.float32),
        pltpu.VMEM((BPW, D), jnp.float32),
        pltpu.VMEM((BPW, 1, D), jnp.float32),
        pltpu.VMEM((BPW, TP_PAD), jnp.int32),
        pltpu.SemaphoreType.DMA,
        pltpu.SemaphoreType.DMA,
        pltpu.SemaphoreType.DMA,
        pltpu.SemaphoreType.DMA,
        pltpu.SemaphoreType.DMA,
        pltpu.SemaphoreType.DMA,
        pltpu.SemaphoreType.DMA,
        pltpu.SemaphoreType.DMA,
        pltpu.SemaphoreType.DMA,
        pltpu.SemaphoreType.DMA,
        pltpu.SemaphoreType.DMA,
        pltpu.SemaphoreType.DMA,
        pltpu.SemaphoreType.DMA,
        pltpu.SemaphoreType.DMA,
    ],
)
def _sc_gather(cls_hbm, ctx2_hbm, pref_hbm, suf2_hbm, tp_hbm,
               out_pT, out_tp,
               idx_v, sb0, sb1, sb2, sb3, sb4, sb5, pref_st, tp_v,
               g0, g1, g2, g3, g4, g5, w0, w1, w2, w3, w4, w5, psem, tsem):
    _body(cls_hbm, ctx2_hbm, pref_hbm, suf2_hbm, tp_hbm, out_pT, out_tp,
          idx_v, sb0, sb1, sb2, sb3, sb4, sb5, pref_st, tp_v,
          g0, g1, g2, g3, g4, g5, w0, w1, w2, w3, w4, w5, psem, tsem)


def kernel(cls_id, ctx, token_prefix, token_suffix, tokenized_prompts):
    cls32 = cls_id.astype(jnp.int32)
    ctx2 = ctx.reshape(N_CLS * N_CTX, D)
    suf2 = jnp.transpose(token_suffix, (1, 0, 2)).reshape(SUF * N_CLS, D)
    tp_pad = jnp.pad(tokenized_prompts, ((0, 0), (0, TP_PAD - SEQ)))
    prompts_T, tp = _sc_gather(cls32, ctx2, token_prefix, suf2, tp_pad)
    return jnp.transpose(prompts_T, (1, 0, 2)), tp[:, :SEQ]
